# baseline (speedup 1.0000x reference)
import functools

import jax
import jax.numpy as jnp
from jax import lax
from jax.experimental import pallas as pl
from jax.experimental.pallas import tpu as pltpu
from jax.experimental.pallas import tpu_sc as plsc

_K = 16
_SCALER = 256.0
_BLK = 512
_F32 = jnp.float32


def _mm(a, b):
    return jax.lax.dot_general(a, b, (((1,), (0,)), ((), ())),
                               preferred_element_type=_F32)


def _make_knn_body(blk, n):
    def body(xyz_ref, xyzT_ref, idx_out_ref):
        q = xyz_ref[0]
        colid = jax.lax.broadcasted_iota(jnp.int32, (blk, n), 1)
        dist = ((q[:, 0:1] - xyzT_ref[0, 0:1, :]) ** 2
                + (q[:, 1:2] - xyzT_ref[0, 1:2, :]) ** 2
                + (q[:, 2:3] - xyzT_ref[0, 2:3, :]) ** 2)
        idxs = []
        minv = None
        for j in range(_K):
            if j == 0:
                idxv = (jax.lax.broadcasted_iota(jnp.int32, (blk, 1), 0)
                        + pl.program_id(1) * blk)
            else:
                cand = jnp.where(dist <= minv, colid, n)
                idxv = jnp.min(cand, axis=1, keepdims=True)
            if j < _K - 1:
                dist = jnp.where(colid == idxv, _F32(3.0e38), dist)
                minv = jnp.min(dist, axis=1, keepdims=True)
            idxs.append(idxv)
        idx_out_ref[0] = jnp.concatenate(idxs, axis=1)
    return body


def _knn_idx(xyz):
    B, n, _ = xyz.shape
    blk = min(512, n)
    xyzT = jnp.pad(jnp.swapaxes(xyz, 1, 2), ((0, 0), (0, 5), (0, 0)))
    return pl.pallas_call(
        _make_knn_body(blk, n),
        grid=(B, n // blk),
        in_specs=[pl.BlockSpec((1, blk, 3), lambda b, i: (b, i, 0)),
                  pl.BlockSpec((1, 8, n), lambda b, i: (b, 0, 0))],
        out_specs=pl.BlockSpec((1, blk, _K), lambda b, i: (b, i, 0)),
        out_shape=jax.ShapeDtypeStruct((B, n, _K), jnp.int32),
    )(xyz, xyzT)


def _sc_gather(src_flat, idx_flat):
    R = idx_flat.shape[0]
    W = src_flat.shape[1]
    info = plsc.get_sparse_core_info()
    nw = info.num_cores * info.num_subcores
    rw = R // nw
    ch = min(rw, 512)
    nch = rw // ch
    mesh = plsc.VectorSubcoreMesh(core_axis_name="c", subcore_axis_name="s")

    @functools.partial(
        pl.kernel, mesh=mesh,
        out_type=jax.ShapeDtypeStruct((R, W), jnp.float32),
        compiler_params=pltpu.CompilerParams(use_tc_tiling_on_sc=False),
        scratch_types=[
            pltpu.VMEM((ch,), jnp.int32),
            pltpu.VMEM((ch,), jnp.int32),
            pltpu.VMEM((ch, W), jnp.float32),
            pltpu.VMEM((ch, W), jnp.float32),
            pltpu.SemaphoreType.DMA,
            pltpu.SemaphoreType.DMA,
            pltpu.SemaphoreType.DMA,
            pltpu.SemaphoreType.DMA,
            pltpu.SemaphoreType.DMA,
            pltpu.SemaphoreType.DMA,
        ],
    )
    def k(src_hbm, idx_hbm, out_hbm, *scr):
        idx_b = scr[0:2]
        rows_b = scr[2:4]
        s_i = scr[4:6]
        s_g = scr[6:8]
        s_w = scr[8:10]
        wid = lax.axis_index("s") * info.num_cores + lax.axis_index("c")
        base = wid * rw
        idx_cp = [None, None]
        wb_cp = [None, None]
        idx_cp[0] = pltpu.async_copy(idx_hbm.at[pl.ds(base, ch)], idx_b[0],
                                     s_i[0])
        for c in range(nch):
            b = c % 2
            if c + 1 < nch:
                idx_cp[1 - b] = pltpu.async_copy(
                    idx_hbm.at[pl.ds(base + (c + 1) * ch, ch)],
                    idx_b[1 - b], s_i[1 - b])
            idx_cp[b].wait()
            if wb_cp[b] is not None:
                wb_cp[b].wait()
            pltpu.async_copy(src_hbm.at[idx_b[b]], rows_b[b], s_g[b]).wait()
            wb_cp[b] = pltpu.async_copy(
                rows_b[b], out_hbm.at[pl.ds(base + c * ch, ch)], s_w[b])
        for w in wb_cp:
            if w is not None:
                w.wait()

    return k(src_flat, idx_flat)


def _make_mlp_body(blk, cin, mid, out, epi, wp):
    def body(*refs):
        i = 0
        xyz_ref = refs[i]; i += 1
        nb_ref = refs[i]; i += 1
        Wa = refs[i][...]; i += 1
        Wb = refs[i][...]; i += 1
        Wd = refs[i][...]; i += 1
        brel = refs[i][...]; i += 1
        Wfea = refs[i][...]; i += 1
        bfea = refs[i][...]; i += 1
        Watt = refs[i][...]; i += 1
        batt = refs[i][...]; i += 1
        Wout = refs[i][...]; i += 1
        bout = refs[i][...]; i += 1
        if epi == "enc":
            Wenc = refs[i][...]; i += 1
            benc = refs[i][...]; i += 1
        elif epi == "dec":
            Wp = refs[i][...]; i += 1
            bp = refs[i][...]; i += 1
        f_out_ref = refs[i]; i += 1
        if epi == "dec":
            coord_out_ref = refs[i]; i += 1

        q = xyz_ref[0]
        q_wa = _mm(q, Wa)
        nbf = nb_ref[0, 0]
        nb_xyz = nbf[:, 0:3]
        nb_fea = nbf[:, 3:3 + cin]
        qT = jnp.broadcast_to(q[None], (_K, blk, 3)).reshape(_K * blk, 3)
        d = qT - nb_xyz
        dist_f = jnp.sqrt(jnp.sum(d * d, axis=1, keepdims=True) + 1e-8)
        qWaT = jnp.broadcast_to(q_wa[None], (_K, blk, mid)).reshape(_K * blk, mid)
        rel_f = jnp.maximum(qWaT + _mm(nb_xyz, Wb) + dist_f * Wd + brel, 0.0)
        nb_f = jnp.maximum(_mm(nb_fea, Wfea) + bfea, 0.0)
        cat = jnp.concatenate([rel_f, nb_f], axis=1)
        score = _mm(cat, Watt) + batt

        scores = [score[j * blk:(j + 1) * blk] for j in range(_K)]
        cats = [cat[j * blk:(j + 1) * blk] for j in range(_K)]
        m = scores[0]
        for s in scores[1:]:
            m = jnp.maximum(m, s)
        ssum, pooled = None, None
        for s, c in zip(scores, cats):
            e = jnp.exp(s - m)
            ssum = e if ssum is None else ssum + e
            pe = e * c
            pooled = pe if pooled is None else pooled + pe
        pooled = pooled / ssum
        f = jnp.maximum(_mm(pooled, Wout) + bout, 0.0)

        if epi == "enc":
            f2 = _mm(f, Wenc) + benc
            f2 = jnp.round(f2 * _SCALER) / _SCALER
            f_out_ref[0] = f2
        elif epi == "dec":
            half = out // 2
            off0 = _mm(f[:, :half], Wp) + bp
            off1 = _mm(f[:, half:], Wp) + bp
            coord_out_ref[0] = jnp.concatenate([q + off0, q + off1], axis=1)
            f_out_ref[0] = f
        else:
            f_out_ref[0] = f
    return body


def _full_spec(shape):
    nd = len(shape)
    return pl.BlockSpec(shape, lambda b, i, _nd=nd: (0,) * _nd)


def _lfa_stage(xyz, fea, p, idx, epi=None, extra=None):
    B, n, _ = xyz.shape
    cin = fea.shape[-1]
    mid = p["Wrel"].shape[1]
    out = p["Wout"].shape[1]
    blk = min(_BLK, n)

    Wr = p["Wrel"]
    Wa = Wr[0:3] + Wr[6:9]
    Wb = Wr[3:6] - Wr[6:9]
    Wd = Wr[9:10]

    W = 3 + cin
    Wp_ = -(-W // 16) * 16
    src = jnp.concatenate([xyz, fea], axis=-1)
    src = jnp.pad(src, ((0, 0), (0, 0), (0, Wp_ - W))).reshape(B * n, Wp_)
    idx_t = idx + jnp.arange(B, dtype=jnp.int32)[:, None, None] * n
    idx_t = idx_t.reshape(B, n // blk, blk, _K).transpose(0, 1, 3, 2)
    nb = _sc_gather(src, idx_t.reshape(B * _K * n))
    nb4 = nb.reshape(B, n // blk, _K * blk, Wp_)

    inputs = [xyz, nb4]
    in_specs = [pl.BlockSpec((1, blk, 3), lambda b, i: (b, i, 0)),
                pl.BlockSpec((1, 1, _K * blk, Wp_), lambda b, i: (b, i, 0, 0))]

    weights = [Wa, Wb, Wd, p["brel"].reshape(1, -1),
               p["Wfea"], p["bfea"].reshape(1, -1),
               p["Watt"], p["batt"].reshape(1, -1),
               p["Wout"], p["bout"].reshape(1, -1)]
    if epi == "enc":
        weights += [extra["W"], extra["b"].reshape(1, -1)]
    elif epi == "dec":
        weights += [extra["Wp"], extra["bp"].reshape(1, -1)]
    for w in weights:
        inputs.append(w)
        in_specs.append(_full_spec(w.shape))

    fdim = extra["W"].shape[1] if epi == "enc" else out
    out_shape = [jax.ShapeDtypeStruct((B, n, fdim), _F32)]
    out_specs = [pl.BlockSpec((1, blk, fdim), lambda b, i: (b, i, 0))]
    if epi == "dec":
        out_shape.append(jax.ShapeDtypeStruct((B, n, 6), _F32))
        out_specs.append(pl.BlockSpec((1, blk, 6), lambda b, i: (b, i, 0)))

    res = pl.pallas_call(
        _make_mlp_body(blk, cin, mid, out, epi, Wp_),
        grid=(B, n // blk),
        in_specs=in_specs,
        out_specs=out_specs,
        out_shape=out_shape,
    )(*inputs)
    return res


def kernel(raw_fea, params):
    xyz = raw_fea[:, :, :3]
    f = raw_fea
    enc = params["enc"]
    for lvl in range(4):
        idx = _knn_idx(xyz)
        (f,) = _lfa_stage(xyz, f, enc[2 * lvl], idx)
        epi = "enc" if lvl == 3 else None
        extra = params["enc_out"] if lvl == 3 else None
        (f,) = _lfa_stage(xyz, f, enc[2 * lvl + 1], idx, epi=epi, extra=extra)
        if lvl < 3:
            xyz, f = xyz[:, ::2], f[:, ::2]

    coord = None
    for di, d in enumerate(params["dec"]):
        if di > 0:
            idx = _knn_idx(xyz)
        f2, coords = _lfa_stage(xyz, f, d["lfa"], idx, epi="dec", extra=d)
        B, M, C = f2.shape
        coord = coords.reshape(B, M, 2, 3).reshape(B, 2 * M, 3)
        f = f2.reshape(B, M, 2, C // 2).reshape(B, 2 * M, C // 2)
        xyz = coord
    return coord


# speedup vs baseline: 1.0029x; 1.0029x over previous
import functools

import jax
import jax.numpy as jnp
from jax import lax
from jax.experimental import pallas as pl
from jax.experimental.pallas import tpu as pltpu
from jax.experimental.pallas import tpu_sc as plsc

_K = 16
_SCALER = 256.0
_BLK = 512
_F32 = jnp.float32


def _mm(a, b):
    return jax.lax.dot_general(a, b, (((1,), (0,)), ((), ())),
                               preferred_element_type=_F32)


def _make_knn_body(blk, n):
    def body(xyz_ref, xyzT_ref, idx_out_ref):
        q = xyz_ref[0]
        colid = jax.lax.broadcasted_iota(jnp.int32, (blk, n), 1)
        dist = ((q[:, 0:1] - xyzT_ref[0, 0:1, :]) ** 2
                + (q[:, 1:2] - xyzT_ref[0, 1:2, :]) ** 2
                + (q[:, 2:3] - xyzT_ref[0, 2:3, :]) ** 2)
        idxs = []
        minv = None
        for j in range(_K):
            if j == 0:
                idxv = (jax.lax.broadcasted_iota(jnp.int32, (blk, 1), 0)
                        + pl.program_id(1) * blk)
            else:
                cand = jnp.where(dist <= minv, colid, n)
                idxv = jnp.min(cand, axis=1, keepdims=True)
            if j < _K - 1:
                dist = jnp.where(colid == idxv, _F32(3.0e38), dist)
                minv = jnp.min(dist, axis=1, keepdims=True)
            idxs.append(idxv)
        idx_out_ref[0] = jnp.concatenate(idxs, axis=1)
    return body


def _knn_idx(xyz):
    B, n, _ = xyz.shape
    blk = min(512, n)
    xyzT = jnp.pad(jnp.swapaxes(xyz, 1, 2), ((0, 0), (0, 5), (0, 0)))
    return pl.pallas_call(
        _make_knn_body(blk, n),
        grid=(B, n // blk),
        in_specs=[pl.BlockSpec((1, blk, 3), lambda b, i: (b, i, 0)),
                  pl.BlockSpec((1, 8, n), lambda b, i: (b, 0, 0))],
        out_specs=pl.BlockSpec((1, blk, _K), lambda b, i: (b, i, 0)),
        out_shape=jax.ShapeDtypeStruct((B, n, _K), jnp.int32),
    )(xyz, xyzT)


def _sc_gather(src_flat, idx_flat):
    R = idx_flat.shape[0]
    W = src_flat.shape[1]
    info = plsc.get_sparse_core_info()
    nw = info.num_cores * info.num_subcores
    rw = R // nw
    ch = min(rw, 1024)
    nch = rw // ch
    mesh = plsc.VectorSubcoreMesh(core_axis_name="c", subcore_axis_name="s")

    @functools.partial(
        pl.kernel, mesh=mesh,
        out_type=jax.ShapeDtypeStruct((R, W), jnp.float32),
        compiler_params=pltpu.CompilerParams(use_tc_tiling_on_sc=False),
        scratch_types=[
            pltpu.VMEM((ch,), jnp.int32),
            pltpu.VMEM((ch, W), jnp.float32),
            pltpu.SemaphoreType.DMA,
        ],
    )
    def k(src_hbm, idx_hbm, out_hbm, idx_v, rows_v, sem):
        wid = lax.axis_index("s") * info.num_cores + lax.axis_index("c")
        for c in range(nch):
            off = wid * rw + c * ch
            pltpu.sync_copy(idx_hbm.at[pl.ds(off, ch)], idx_v)
            pltpu.async_copy(src_hbm.at[idx_v], rows_v, sem).wait()
            pltpu.sync_copy(rows_v, out_hbm.at[pl.ds(off, ch)])

    return k(src_flat, idx_flat)


def _make_mlp_body(blk, cin, mid, out, epi, wp):
    def body(*refs):
        i = 0
        xyz_ref = refs[i]; i += 1
        nb_ref = refs[i]; i += 1
        Wa = refs[i][...]; i += 1
        Wb = refs[i][...]; i += 1
        Wd = refs[i][...]; i += 1
        brel = refs[i][...]; i += 1
        Wfea = refs[i][...]; i += 1
        bfea = refs[i][...]; i += 1
        Watt = refs[i][...]; i += 1
        batt = refs[i][...]; i += 1
        Wout = refs[i][...]; i += 1
        bout = refs[i][...]; i += 1
        if epi == "enc":
            Wenc = refs[i][...]; i += 1
            benc = refs[i][...]; i += 1
        elif epi == "dec":
            Wp = refs[i][...]; i += 1
            bp = refs[i][...]; i += 1
        f_out_ref = refs[i]; i += 1
        if epi == "dec":
            coord_out_ref = refs[i]; i += 1

        q = xyz_ref[0]
        q_wa = _mm(q, Wa)
        nbf = nb_ref[0, 0]
        nb_xyz = nbf[:, 0:3]
        nb_fea = nbf[:, 3:3 + cin]
        qT = jnp.broadcast_to(q[None], (_K, blk, 3)).reshape(_K * blk, 3)
        d = qT - nb_xyz
        dist_f = jnp.sqrt(jnp.sum(d * d, axis=1, keepdims=True) + 1e-8)
        qWaT = jnp.broadcast_to(q_wa[None], (_K, blk, mid)).reshape(_K * blk, mid)
        rel_f = jnp.maximum(qWaT + _mm(nb_xyz, Wb) + dist_f * Wd + brel, 0.0)
        nb_f = jnp.maximum(_mm(nb_fea, Wfea) + bfea, 0.0)
        cat = jnp.concatenate([rel_f, nb_f], axis=1)
        score = _mm(cat, Watt) + batt

        scores = [score[j * blk:(j + 1) * blk] for j in range(_K)]
        cats = [cat[j * blk:(j + 1) * blk] for j in range(_K)]
        m = scores[0]
        for s in scores[1:]:
            m = jnp.maximum(m, s)
        ssum, pooled = None, None
        for s, c in zip(scores, cats):
            e = jnp.exp(s - m)
            ssum = e if ssum is None else ssum + e
            pe = e * c
            pooled = pe if pooled is None else pooled + pe
        pooled = pooled / ssum
        f = jnp.maximum(_mm(pooled, Wout) + bout, 0.0)

        if epi == "enc":
            f2 = _mm(f, Wenc) + benc
            f2 = jnp.round(f2 * _SCALER) / _SCALER
            f_out_ref[0] = f2
        elif epi == "dec":
            half = out // 2
            off0 = _mm(f[:, :half], Wp) + bp
            off1 = _mm(f[:, half:], Wp) + bp
            coord_out_ref[0] = jnp.concatenate([q + off0, q + off1], axis=1)
            f_out_ref[0] = f
        else:
            f_out_ref[0] = f
    return body


def _full_spec(shape):
    nd = len(shape)
    return pl.BlockSpec(shape, lambda b, i, _nd=nd: (0,) * _nd)


def _lfa_stage(xyz, fea, p, idx, epi=None, extra=None):
    B, n, _ = xyz.shape
    cin = fea.shape[-1]
    mid = p["Wrel"].shape[1]
    out = p["Wout"].shape[1]
    blk = min(_BLK, n)

    Wr = p["Wrel"]
    Wa = Wr[0:3] + Wr[6:9]
    Wb = Wr[3:6] - Wr[6:9]
    Wd = Wr[9:10]

    W = 3 + cin
    Wp_ = -(-W // 16) * 16
    src = jnp.concatenate([xyz, fea], axis=-1)
    src = jnp.pad(src, ((0, 0), (0, 0), (0, Wp_ - W))).reshape(B * n, Wp_)
    idx_t = idx + jnp.arange(B, dtype=jnp.int32)[:, None, None] * n
    idx_t = idx_t.reshape(B, n // blk, blk, _K).transpose(0, 1, 3, 2)
    nb = _sc_gather(src, idx_t.reshape(B * _K * n))
    nb4 = nb.reshape(B, n // blk, _K * blk, Wp_)

    inputs = [xyz, nb4]
    in_specs = [pl.BlockSpec((1, blk, 3), lambda b, i: (b, i, 0)),
                pl.BlockSpec((1, 1, _K * blk, Wp_), lambda b, i: (b, i, 0, 0))]

    weights = [Wa, Wb, Wd, p["brel"].reshape(1, -1),
               p["Wfea"], p["bfea"].reshape(1, -1),
               p["Watt"], p["batt"].reshape(1, -1),
               p["Wout"], p["bout"].reshape(1, -1)]
    if epi == "enc":
        weights += [extra["W"], extra["b"].reshape(1, -1)]
    elif epi == "dec":
        weights += [extra["Wp"], extra["bp"].reshape(1, -1)]
    for w in weights:
        inputs.append(w)
        in_specs.append(_full_spec(w.shape))

    fdim = extra["W"].shape[1] if epi == "enc" else out
    out_shape = [jax.ShapeDtypeStruct((B, n, fdim), _F32)]
    out_specs = [pl.BlockSpec((1, blk, fdim), lambda b, i: (b, i, 0))]
    if epi == "dec":
        out_shape.append(jax.ShapeDtypeStruct((B, n, 6), _F32))
        out_specs.append(pl.BlockSpec((1, blk, 6), lambda b, i: (b, i, 0)))

    res = pl.pallas_call(
        _make_mlp_body(blk, cin, mid, out, epi, Wp_),
        grid=(B, n // blk),
        in_specs=in_specs,
        out_specs=out_specs,
        out_shape=out_shape,
    )(*inputs)
    return res


def kernel(raw_fea, params):
    xyz = raw_fea[:, :, :3]
    f = raw_fea
    enc = params["enc"]
    for lvl in range(4):
        idx = _knn_idx(xyz)
        (f,) = _lfa_stage(xyz, f, enc[2 * lvl], idx)
        epi = "enc" if lvl == 3 else None
        extra = params["enc_out"] if lvl == 3 else None
        (f,) = _lfa_stage(xyz, f, enc[2 * lvl + 1], idx, epi=epi, extra=extra)
        if lvl < 3:
            xyz, f = xyz[:, ::2], f[:, ::2]

    coord = None
    for di, d in enumerate(params["dec"]):
        if di > 0:
            idx = _knn_idx(xyz)
        f2, coords = _lfa_stage(xyz, f, d["lfa"], idx, epi="dec", extra=d)
        B, M, C = f2.shape
        coord = coords.reshape(B, M, 2, 3).reshape(B, 2 * M, 3)
        f = f2.reshape(B, M, 2, C // 2).reshape(B, 2 * M, C // 2)
        xyz = coord
    return coord
